# Initial kernel scaffold; baseline (speedup 1.0000x reference)
import jax
import jax.numpy as jnp
from jax.experimental import pallas as pl


def kernel(x, edge_index, review_feat, edge_w, node_W, review_W):
    raise NotImplementedError("write your pallas kernel here")


# speedup vs baseline: 2.3138x; 2.3138x over previous
import functools

import jax
import jax.numpy as jnp
from jax import lax
from jax.experimental import pallas as pl
from jax.experimental.pallas import tpu as pltpu
from jax.experimental.pallas import tpu_sc as plsc

N = 50000
E = 800000
D_NODE = 16
D_REV = 64

NC = 2
NS = 16
NW = NC * NS

NPAD = 50048

SUB = 128
CH = 8
CHUNK = SUB * CH
EPAD = 819200
PER_W = EPAD // NW
NCHUNK = PER_W // CHUNK
ROWS_PER_TILE = NPAD // NS

BE = 6400
REAL_BLOCKS = E // BE


def _h_body(x_ref, w_ref, o_ref):
    o_ref[...] = jnp.dot(x_ref[...], w_ref[...], preferred_element_type=jnp.float32)


def _rfw_body(rf_ref, w_ref, ew_ref, o_ref):
    o_ref[...] = (
        jnp.dot(rf_ref[...], w_ref[...], preferred_element_type=jnp.float32)
        * ew_ref[...]
    )


def _add_body(a_ref, b_ref, o_ref):
    o_ref[...] = a_ref[...] + b_ref[...]


_sc_mesh = plsc.VectorSubcoreMesh(core_axis_name="c", subcore_axis_name="s")


@functools.partial(
    pl.kernel,
    out_type=jax.ShapeDtypeStruct((NC, NPAD, D_NODE), jnp.float32),
    mesh=_sc_mesh,
    scratch_types=[
        pltpu.VMEM((CH, SUB), jnp.int32),
        pltpu.VMEM((CH, SUB), jnp.int32),
        pltpu.VMEM((CHUNK,), jnp.float32),
        pltpu.VMEM((CHUNK, D_NODE), jnp.float32),
        pltpu.VMEM((CHUNK, D_NODE), jnp.float32),
        pltpu.VMEM_SHARED((NPAD, D_NODE), jnp.float32),
        pltpu.SemaphoreType.DMA,
    ],
    compiler_params=pltpu.CompilerParams(use_tc_tiling_on_sc=False),
)
def _sc_scatter(h_hbm, rfw_hbm, src_hbm, dst_hbm, w_hbm, zeros_hbm, out_hbm,
                sidx_v, didx_v, w_v, g_v, rfw_v, acc, sem):
    cid = lax.axis_index("c")
    sid = lax.axis_index("s")
    wid = cid * NS + sid

    pltpu.sync_copy(zeros_hbm, acc.at[pl.ds(sid * ROWS_PER_TILE, ROWS_PER_TILE)])
    plsc.subcore_barrier()

    base_row = wid * (PER_W // SUB)
    base_e = wid * PER_W

    for g in range(NCHUNK):
        row0 = base_row + g * CH
        e0 = base_e + g * CHUNK
        pltpu.sync_copy(src_hbm.at[pl.ds(row0, CH)], sidx_v)
        pltpu.sync_copy(dst_hbm.at[pl.ds(row0, CH)], didx_v)
        pltpu.sync_copy(w_hbm.at[pl.ds(e0, CHUNK)], w_v)
        pltpu.sync_copy(rfw_hbm.at[pl.ds(e0, CHUNK)], rfw_v)
        descs = [
            pltpu.async_copy(h_hbm.at[sidx_v.at[j]],
                             g_v.at[pl.ds(j * SUB, SUB)], sem)
            for j in range(CH)
        ]
        for d in descs:
            d.wait()

        def body(k, _):
            i0 = k * 16
            wvec = w_v[pl.ds(i0, 16)]
            for j in range(16):
                i = i0 + j
                g_v[i, :] = g_v[i, :] * wvec[j] + rfw_v[i, :]
            return 0

        lax.fori_loop(0, CHUNK // 16, body, 0)

        for j in range(CH):
            pltpu.sync_copy(g_v.at[pl.ds(j * SUB, SUB)],
                            acc.at[didx_v.at[j]], add=True)

    plsc.subcore_barrier()
    pltpu.sync_copy(acc.at[pl.ds(sid * ROWS_PER_TILE, ROWS_PER_TILE)],
                    out_hbm.at[cid, pl.ds(sid * ROWS_PER_TILE, ROWS_PER_TILE)])


def kernel(x, edge_index, review_feat, edge_w, node_W, review_W):
    src = edge_index[0]
    dst = edge_index[1]
    w = edge_w[:, 0]

    pad = EPAD - E
    src_p = jnp.concatenate([src, jnp.zeros((pad,), jnp.int32)]).reshape(EPAD // SUB, SUB)
    dst_p = jnp.concatenate([dst, jnp.full((pad,), N, jnp.int32)]).reshape(EPAD // SUB, SUB)
    w_p = jnp.concatenate([w, jnp.zeros((pad,), jnp.float32)])
    ew_p = jnp.concatenate([edge_w, jnp.zeros((pad, 1), jnp.float32)])
    zeros = jnp.zeros((ROWS_PER_TILE, D_NODE), jnp.float32)

    h = pl.pallas_call(
        _h_body,
        out_shape=jax.ShapeDtypeStruct((N, D_NODE), jnp.float32),
    )(x, node_W)

    rfw = pl.pallas_call(
        _rfw_body,
        grid=(EPAD // BE,),
        in_specs=[
            pl.BlockSpec((BE, D_REV), lambda i: (jnp.minimum(i, REAL_BLOCKS - 1), 0)),
            pl.BlockSpec((D_REV, D_NODE), lambda i: (0, 0)),
            pl.BlockSpec((BE, 1), lambda i: (i, 0)),
        ],
        out_specs=pl.BlockSpec((BE, D_NODE), lambda i: (i, 0)),
        out_shape=jax.ShapeDtypeStruct((EPAD, D_NODE), jnp.float32),
    )(review_feat, review_W, ew_p)

    partials = _sc_scatter(h, rfw, src_p, dst_p, w_p, zeros)

    rst = pl.pallas_call(
        _add_body,
        grid=(10,),
        in_specs=[
            pl.BlockSpec((N // 10, D_NODE), lambda i: (i, 0)),
            pl.BlockSpec((N // 10, D_NODE), lambda i: (i, 0)),
        ],
        out_specs=pl.BlockSpec((N // 10, D_NODE), lambda i: (i, 0)),
        out_shape=jax.ShapeDtypeStruct((N, D_NODE), jnp.float32),
    )(partials[0], partials[1])
    return rst
